# Initial kernel scaffold; baseline (speedup 1.0000x reference)
import jax
import jax.numpy as jnp
from jax.experimental import pallas as pl


def kernel(x_src, x_dst, edge_index, edge_weight, W_loop, W_intr):
    raise NotImplementedError("write your pallas kernel here")


# speedup vs baseline: 5.3864x; 5.3864x over previous
import functools

import jax
import jax.numpy as jnp
from jax import lax
from jax.experimental import pallas as pl
from jax.experimental.pallas import tpu as pltpu
from jax.experimental.pallas import tpu_sc as plsc

N_NODES = 10000
D = 256
DH = 128
N_TILES = 16
NB = 79
KB = 128
E_PAD = N_TILES * NB * KB
N_ACC = 10240
ROWS_PER_TILE = N_ACC // N_TILES
ZCHUNKS = ROWS_PER_TILE // KB

_mesh = plsc.VectorSubcoreMesh(core_axis_name="c", subcore_axis_name="s")


@functools.partial(
    pl.kernel,
    out_type=jax.ShapeDtypeStruct((2 * N_ACC, DH), jnp.float32),
    mesh=_mesh,
    scratch_types=[
        pltpu.VMEM((NB, KB), jnp.int32),
        pltpu.VMEM((NB, KB), jnp.int32),
        pltpu.VMEM((NB, KB), jnp.float32),
        pltpu.VMEM((KB, DH), jnp.float32),
        pltpu.VMEM_SHARED((N_ACC, DH), jnp.float32),
        pltpu.SemaphoreType.DMA,
    ],
)
def _sc_segment(xs_hbm, isrc_hbm, idst_hbm, w_hbm, out_hbm,
                isrc_v, idst_v, w_v, rows_v, acc, sem):
    c = lax.axis_index("c")
    s = lax.axis_index("s")

    pltpu.sync_copy(isrc_hbm.at[s], isrc_v)
    pltpu.sync_copy(idst_hbm.at[s], idst_v)
    pltpu.sync_copy(w_hbm.at[s], w_v)

    off = jnp.broadcast_to((c * N_NODES).astype(jnp.int32), (16,))

    @pl.loop(0, NB)
    def _adj(b):
        for j in range(KB // 16):
            sl = pl.ds(16 * j, 16)
            isrc_v[b, sl] = isrc_v[b, sl] + off

    zero = jnp.zeros((16,), jnp.float32)

    @pl.loop(0, KB)
    def _zr(e):
        for r in range(DH // 16):
            rows_v[e, pl.ds(16 * r, 16)] = zero

    for j in range(ZCHUNKS):
        pltpu.sync_copy(rows_v, acc.at[pl.ds((s * ZCHUNKS + j) * KB, KB)])
    plsc.subcore_barrier()

    @pl.loop(0, NB)
    def _body(b):
        pltpu.async_copy(xs_hbm.at[isrc_v.at[b]], rows_v, sem).wait()

        @pl.loop(0, KB // 16)
        def _scale(g):
            wvec = w_v[b, pl.ds(16 * g, 16)]
            for j in range(16):
                wj = jnp.broadcast_to(wvec[j], (16,))
                e = 16 * g + j
                for r in range(DH // 16):
                    sl = pl.ds(16 * r, 16)
                    rows_v[e, sl] = rows_v[e, sl] * wj

        pltpu.sync_copy(rows_v, acc.at[idst_v.at[b]], add=True)

    plsc.subcore_barrier()

    base = c * N_ACC + s * ROWS_PER_TILE
    pltpu.sync_copy(acc.at[pl.ds(s * ROWS_PER_TILE, ROWS_PER_TILE)],
                    out_hbm.at[pl.ds(base, ROWS_PER_TILE)])


def _tc_body(xd_ref, g_ref, wlt_ref, wit_ref, out_ref):
    xd = xd_ref[...]
    g = g_ref[...]
    y = jnp.dot(xd + g, wlt_ref[...], preferred_element_type=jnp.float32)
    y += jnp.dot(xd * g, wit_ref[...], preferred_element_type=jnp.float32)
    out_ref[...] = jnp.where(y >= 0, y, 0.01 * y)


_TR = 512


def _tc_post(xd_pad, g_pad, wlt, wit):
    return pl.pallas_call(
        _tc_body,
        grid=(N_ACC // _TR,),
        in_specs=[
            pl.BlockSpec((_TR, D), lambda i: (i, 0)),
            pl.BlockSpec((_TR, D), lambda i: (i, 0)),
            pl.BlockSpec((D, D), lambda i: (0, 0)),
            pl.BlockSpec((D, D), lambda i: (0, 0)),
        ],
        out_specs=pl.BlockSpec((_TR, D), lambda i: (i, 0)),
        out_shape=jax.ShapeDtypeStruct((N_ACC, D), jnp.float32),
    )(xd_pad, g_pad, wlt, wit)


@jax.jit
def kernel(x_src, x_dst, edge_index, edge_weight, W_loop, W_intr):
    E = edge_index.shape[1]
    i_src = edge_index[0].astype(jnp.int32)
    i_dst = edge_index[1].astype(jnp.int32)
    w = edge_weight[:, 0]

    pad = E_PAD - E
    i_src_p = jnp.pad(i_src, (0, pad)).reshape(N_TILES, NB, KB)
    i_dst_p = jnp.pad(i_dst, (0, pad)).reshape(N_TILES, NB, KB)
    w_p = jnp.pad(w, (0, pad)).reshape(N_TILES, NB, KB)

    xs = jnp.concatenate([x_src[:, :DH], x_src[:, DH:]], axis=0)

    out = _sc_segment(xs, i_src_p, i_dst_p, w_p)
    g = jnp.concatenate(
        [out[:N_NODES], out[N_ACC:N_ACC + N_NODES]], axis=1)

    g_pad = jnp.pad(g, ((0, N_ACC - N_NODES), (0, 0)))
    xd_pad = jnp.pad(x_dst, ((0, N_ACC - N_NODES), (0, 0)))
    res = _tc_post(xd_pad, g_pad, W_loop.T, W_intr.T)
    return res[:N_NODES]
